# baseline (speedup 1.0000x reference)
import jax
import jax.numpy as jnp
from jax.experimental import pallas as pl
from jax.experimental.pallas import tpu as pltpu

_ROWS = 16384
_COLS = 200
_BR = 2048


def _tc_body(p_ref, t_ref, m_ref, out_ref, acc_ref):
    i = pl.program_id(0)

    @pl.when(i == 0)
    def _init():
        acc_ref[0] = 0.0
        acc_ref[1] = 0.0

    p = p_ref[...]
    t = t_ref[...]
    m = m_ref[...]
    elem = jnp.exp(p) - t * p
    acc_ref[0] += jnp.sum(jnp.where(m, elem, 0.0))
    acc_ref[1] += jnp.sum(m.astype(jnp.float32))

    @pl.when(i == pl.num_programs(0) - 1)
    def _fin():
        out_ref[0, 0] = acc_ref[0] / acc_ref[1]


@jax.jit
def kernel(y_pred, y_true, mask):
    out = pl.pallas_call(
        _tc_body,
        grid=(_ROWS // _BR,),
        in_specs=[
            pl.BlockSpec((_BR, _COLS), lambda i: (i, 0)),
            pl.BlockSpec((_BR, _COLS), lambda i: (i, 0)),
            pl.BlockSpec((_BR, _COLS), lambda i: (i, 0)),
        ],
        out_specs=pl.BlockSpec(memory_space=pltpu.SMEM),
        out_shape=jax.ShapeDtypeStruct((1, 1), jnp.float32),
        scratch_shapes=[pltpu.SMEM((2,), jnp.float32)],
    )(y_pred, y_true, mask)
    return out[0, 0]
